# baseline (speedup 1.0000x reference)
import functools
import jax
import jax.numpy as jnp
from jax import lax
from jax.experimental import pallas as pl
from jax.experimental.pallas import tpu as pltpu
from jax.experimental.pallas import tpu_sc as plsc

BB, NN = 8, 8192
NRF = 1024
GG = BB * NRF
MAXEV = 32
ECH = 512
NCH = NN // ECH
NW = 32
EPW = BB * NN // NW
IDXW = 128
NIDX = EPW // IDXW
ROWS = GG * MAXEV + BB * NN


def _binning_body(cxr, cyr, vr, tsr, xr,
                  dest_ref, f0_ref, f1_ref, f2_ref, f3_ref, cnt_ref,
                  rank_s, delay_s):
    b = pl.program_id(0)
    cols = lax.broadcasted_iota(jnp.int32, (ECH, NRF), 1)
    ri = lax.broadcasted_iota(jnp.int32, (ECH, ECH), 0)
    ci = lax.broadcasted_iota(jnp.int32, (ECH, ECH), 1)
    tri = ci < ri
    ieq = ci == ri

    def to_col(rowv):
        return jnp.sum(jnp.where(ieq, rowv, 0), axis=1, keepdims=True)

    def to_row(colv):
        return jnp.sum(jnp.where(ieq, colv, 0), axis=0, keepdims=True)

    def rf_row(c):
        sl = pl.ds(c * ECH, ECH)
        rf = (cyr[0, :, sl] // 2) * 32 + (cxr[0, :, sl] // 2)
        return jnp.where(vr[0, :, sl] > 0.5, rf, NRF)

    def ph1(c, carry):
        cnt, last_ts = carry
        sl = pl.ds(c * ECH, ECH)
        rf_r = rf_row(c)
        rf_c = to_col(rf_r)
        ts_r = tsr[0, :, sl]
        ts_c = to_col(ts_r)
        oh = rf_c == cols
        m = (rf_c == rf_r) & tri
        lrank = jnp.sum(m.astype(jnp.float32), axis=1, keepdims=True)
        prev_chunk = jnp.max(jnp.where(m, ts_r, -1.0), axis=1, keepdims=True)
        rank_base = jnp.max(jnp.where(oh, cnt, 0.0), axis=1, keepdims=True)
        prev_carry = jnp.max(jnp.where(oh, last_ts, 0.0), axis=1, keepdims=True)
        rank = rank_base + lrank
        prev = jnp.where(lrank > 0, prev_chunk, prev_carry)
        delay = jnp.where(rank > 0, ts_c - prev, 0.0)
        rank_s[:, sl] = to_row(rank)
        delay_s[:, sl] = to_row(delay)
        cnt = cnt + jnp.sum(oh.astype(jnp.float32), axis=0, keepdims=True)
        last_ts = jnp.maximum(
            last_ts, jnp.max(jnp.where(oh, ts_c, 0.0), axis=0, keepdims=True))
        return cnt, last_ts

    zrf = jnp.zeros((1, NRF), jnp.float32)
    cnt, _ = lax.fori_loop(0, NCH, ph1, (zrf, zrf))
    shift = cnt - jnp.minimum(cnt, float(MAXEV))

    def ph2(c, maxd):
        sl = pl.ds(c * ECH, ECH)
        rf_c = to_col(rf_row(c))
        oh = rf_c == cols
        shift_e = jnp.max(jnp.where(oh, shift, 0.0), axis=1, keepdims=True)
        nrank = to_col(rank_s[:, sl]) - shift_e
        keep = (nrank >= 0) & (to_col(vr[0, :, sl]) > 0.5)
        mx = jnp.max(jnp.where(oh & keep, to_col(delay_s[:, sl]), 0.0),
                     axis=0, keepdims=True)
        return jnp.maximum(maxd, mx)

    maxd = lax.fori_loop(0, NCH, ph2, zrf)

    def ph3(c, _):
        sl = pl.ds(c * ECH, ECH)
        cxv = cxr[0, :, sl]
        cyv = cyr[0, :, sl]
        rf_r = rf_row(c)
        rf_c = to_col(rf_r)
        oh = rf_c == cols
        shift_e = jnp.max(jnp.where(oh, shift, 0.0), axis=1, keepdims=True)
        maxd_e = jnp.max(jnp.where(oh, maxd, 0.0), axis=1, keepdims=True)
        nrank = rank_s[:, sl] - to_row(shift_e)
        keep = (nrank >= 0) & (vr[0, :, sl] > 0.5)
        nd = delay_s[:, sl] / jnp.maximum(to_row(maxd_e), 1e-9)
        e_idx = lax.broadcasted_iota(jnp.int32, (1, ECH), 1)
        trash = GG * MAXEV + b * NN + c * ECH + e_idx
        dst = jnp.where(keep,
                        (b * NRF + rf_r) * MAXEV + nrank.astype(jnp.int32),
                        trash)
        dest_ref[0, :, sl] = dst
        f0_ref[0, :, sl] = xr[0, :, sl]
        f1_ref[0, :, sl] = (cxv - (cxv // 2) * 2).astype(jnp.float32)
        f2_ref[0, :, sl] = (cyv - (cyv // 2) * 2).astype(jnp.float32)
        f3_ref[0, :, sl] = nd
        return 0

    lax.fori_loop(0, NCH, ph3, 0)
    cnt_ref[0, :, :] = jnp.minimum(cnt, float(MAXEV))


def _binning(cxr, cyr, vr, tsr, xr):
    row = pl.BlockSpec((1, 1, NN), lambda i: (i, 0, 0))
    return pl.pallas_call(
        _binning_body,
        grid=(BB,),
        in_specs=[row, row, row, row, row],
        out_specs=[row, row, row, row, row,
                   pl.BlockSpec((1, 1, NRF), lambda i: (i, 0, 0))],
        out_shape=[
            jax.ShapeDtypeStruct((BB, 1, NN), jnp.int32),
            jax.ShapeDtypeStruct((BB, 1, NN), jnp.float32),
            jax.ShapeDtypeStruct((BB, 1, NN), jnp.float32),
            jax.ShapeDtypeStruct((BB, 1, NN), jnp.float32),
            jax.ShapeDtypeStruct((BB, 1, NN), jnp.float32),
            jax.ShapeDtypeStruct((BB, 1, NRF), jnp.float32),
        ],
        scratch_shapes=[
            pltpu.VMEM((1, NN), jnp.float32),
            pltpu.VMEM((1, NN), jnp.float32),
        ],
        compiler_params=pltpu.CompilerParams(
            dimension_semantics=("parallel",)),
    )(cxr, cyr, vr, tsr, xr)


def _scatter_feats(dest, f0, f1, f2, f3):
    mesh = plsc.VectorSubcoreMesh(core_axis_name="c", subcore_axis_name="s")
    out_t = jax.ShapeDtypeStruct((ROWS,), jnp.float32)

    @functools.partial(
        pl.kernel,
        mesh=mesh,
        out_type=[out_t, out_t, out_t, out_t],
        scratch_types=[
            pltpu.VMEM((NIDX, IDXW), jnp.int32),
            pltpu.VMEM((NIDX, IDXW), jnp.float32),
            pltpu.VMEM((NIDX, IDXW), jnp.float32),
            pltpu.VMEM((NIDX, IDXW), jnp.float32),
            pltpu.VMEM((NIDX, IDXW), jnp.float32),
            pltpu.SemaphoreType.DMA,
        ],
    )
    def k(dest_hbm, f0_hbm, f1_hbm, f2_hbm, f3_hbm,
          o0_hbm, o1_hbm, o2_hbm, o3_hbm,
          idx_v, v0, v1, v2, v3, sem):
        wid = lax.axis_index("s") * 2 + lax.axis_index("c")
        pltpu.sync_copy(dest_hbm.at[wid], idx_v)
        pltpu.sync_copy(f0_hbm.at[wid], v0)
        pltpu.sync_copy(f1_hbm.at[wid], v1)
        pltpu.sync_copy(f2_hbm.at[wid], v2)
        pltpu.sync_copy(f3_hbm.at[wid], v3)

        def body(j, _):
            c0 = pltpu.async_copy(v0.at[j], o0_hbm.at[idx_v.at[j]], sem)
            c1 = pltpu.async_copy(v1.at[j], o1_hbm.at[idx_v.at[j]], sem)
            c2 = pltpu.async_copy(v2.at[j], o2_hbm.at[idx_v.at[j]], sem)
            c3 = pltpu.async_copy(v3.at[j], o3_hbm.at[idx_v.at[j]], sem)
            c0.wait()
            c1.wait()
            c2.wait()
            c3.wait()
            return 0

        lax.fori_loop(0, NIDX, body, 0)

    return k(dest, f0, f1, f2, f3)


def _lstm_body(d0, d1, d2, d3, cnt_ref, wx_ref, wh_ref, bias_ref, out_ref):
    gb = out_ref.shape[0]
    wh = wh_ref[...]
    bias = bias_ref[...]
    lens = cnt_ref[0]
    w0 = wx_ref[0:1, :]
    w1 = wx_ref[1:2, :]
    w2 = wx_ref[2:3, :]
    w3 = wx_ref[3:4, :]
    h = jnp.zeros((gb, 16), jnp.float32)
    c = jnp.zeros((gb, 16), jnp.float32)
    for t in range(MAXEV):
        mask = lens > float(t)
        xg = (d0[:, t:t + 1] * w0 + d1[:, t:t + 1] * w1
              + d2[:, t:t + 1] * w2 + d3[:, t:t + 1] * w3)
        gates = (xg + jnp.dot(h, wh, preferred_element_type=jnp.float32)
                 + bias)
        i = 1.0 / (1.0 + jnp.exp(-gates[:, 0:16]))
        f = 1.0 / (1.0 + jnp.exp(-gates[:, 16:32]))
        g = jnp.tanh(gates[:, 32:48])
        o = 1.0 / (1.0 + jnp.exp(-gates[:, 48:64]))
        c_new = f * c + i * g
        h_new = o * jnp.tanh(c_new)
        h = jnp.where(mask, h_new, h)
        c = jnp.where(mask, c_new, c)
    out_ref[...] = h


def _lstm(t0, t1, t2, t3, cnt3, wx8, wh, bias):
    gblk = 1024
    dsp = pl.BlockSpec((gblk, MAXEV), lambda i: (i, 0))
    return pl.pallas_call(
        _lstm_body,
        grid=(GG // gblk,),
        in_specs=[
            dsp, dsp, dsp, dsp,
            pl.BlockSpec((1, gblk, 1), lambda i: (i, 0, 0)),
            pl.BlockSpec((8, 64), lambda i: (0, 0)),
            pl.BlockSpec((16, 64), lambda i: (0, 0)),
            pl.BlockSpec((1, 64), lambda i: (0, 0)),
        ],
        out_specs=pl.BlockSpec((gblk, 16), lambda i: (i, 0)),
        out_shape=jax.ShapeDtypeStruct((GG, 16), jnp.float32),
        compiler_params=pltpu.CompilerParams(
            dimension_semantics=("parallel",)),
    )(t0, t1, t2, t3, cnt3, wx8, wh, bias)


def kernel(x, coords, ts, lengths, W_ih, W_hh, b_ih, b_hh):
    cx = coords[..., 0].astype(jnp.int32)
    cy = coords[..., 1].astype(jnp.int32)
    valid = (jnp.arange(NN)[None, :] < lengths[:, None]).astype(jnp.float32)
    xs = x[..., 0]
    tss = ts[..., 0]
    row = lambda a: a.reshape(BB, 1, NN)
    dest, f0, f1, f2, f3, cnt = _binning(
        row(cx), row(cy), row(valid), row(tss), row(xs))
    wrk = lambda a: a.reshape(NW, NIDX, IDXW)
    t0, t1, t2, t3 = _scatter_feats(
        wrk(dest), wrk(f0), wrk(f1), wrk(f2), wrk(f3))
    dtab = lambda a: a[:GG * MAXEV].reshape(GG, MAXEV)
    wx8 = jnp.pad(W_ih.T, ((0, 4), (0, 0)))
    bias = (b_ih + b_hh).reshape(1, 64)
    h = _lstm(dtab(t0), dtab(t1), dtab(t2), dtab(t3),
              cnt.reshape(BB, NRF, 1), wx8, W_hh.T, bias)
    return h.reshape(BB, 32, 32, 16)


# speedup vs baseline: 1.2938x; 1.2938x over previous
import functools
import jax
import jax.numpy as jnp
from jax import lax
from jax.experimental import pallas as pl
from jax.experimental.pallas import tpu as pltpu
from jax.experimental.pallas import tpu_sc as plsc

BB, NN = 8, 8192
NRF = 1024
GG = BB * NRF
MAXEV = 32
TSLOT = MAXEV + 1
ECH = 512
NCH = NN // ECH
NW = 32
EPW = BB * NN // NW
IDXW = 128
NIDX = EPW // IDXW
ROWS3 = GG * MAXEV + BB * NN
ROWS33 = GG * TSLOT + BB * NN


def _binning_body(cxr, cyr, vr,
                  dest3_ref, dest33_ref, rel_ref, cnt_ref, rank_s):
    b = pl.program_id(0)
    i32c = lax.broadcasted_iota(jnp.int32, (ECH, 32), 1)
    i32r = lax.broadcasted_iota(jnp.int32, (32, ECH), 0)
    ri = lax.broadcasted_iota(jnp.int32, (ECH, ECH), 0)
    ci = lax.broadcasted_iota(jnp.int32, (ECH, ECH), 1)
    tri = ci < ri
    ieq = ci == ri

    def to_col(rowv):
        return jnp.sum(jnp.where(ieq, rowv, 0), axis=1, keepdims=True)

    def to_row(colv):
        return jnp.sum(jnp.where(ieq, colv, 0), axis=0, keepdims=True)

    def rf_row(c):
        sl = pl.ds(c * ECH, ECH)
        rf = (cyr[0, :, sl] // 2) * 32 + (cxr[0, :, sl] // 2)
        return jnp.where(vr[0, :, sl] > 0.5, rf, NRF)

    def gath(tab, hi_c, lo_c):
        mhi = (hi_c == i32c).astype(jnp.float32)
        ze = jnp.dot(mhi, tab, preferred_element_type=jnp.float32)
        return jnp.sum(jnp.where(lo_c == i32c, ze, 0.0), axis=1, keepdims=True)

    def ph1(c, cnt):
        rf_r = rf_row(c)
        rf_c = to_col(rf_r)
        hi_c = rf_c // 32
        lo_c = rf_c % 32
        m = (rf_c == rf_r) & tri
        lrank = jnp.sum(m.astype(jnp.float32), axis=1, keepdims=True)
        rank = gath(cnt, hi_c, lo_c) + lrank
        rank_s[:, pl.ds(c * ECH, ECH)] = to_row(rank)
        mhit = (i32r == rf_r // 32).astype(jnp.float32)
        mlo = (lo_c == i32c).astype(jnp.float32)
        return cnt + jnp.dot(mhit, mlo, preferred_element_type=jnp.float32)

    cnt = lax.fori_loop(0, NCH, ph1, jnp.zeros((32, 32), jnp.float32))
    shift32 = cnt - jnp.minimum(cnt, float(MAXEV))
    shift33 = cnt - jnp.minimum(cnt, float(TSLOT))

    def ph2(c, _):
        sl = pl.ds(c * ECH, ECH)
        rf_r = rf_row(c)
        rf_c = to_col(rf_r)
        hi_c = rf_c // 32
        lo_c = rf_c % 32
        s32e = to_row(gath(shift32, hi_c, lo_c))
        s33e = to_row(gath(shift33, hi_c, lo_c))
        rank_r = rank_s[:, sl]
        vmask = vr[0, :, sl] > 0.5
        n32 = rank_r - s32e
        n33 = rank_r - s33e
        k32 = (n32 >= 0) & vmask
        k33 = (n33 >= 0) & vmask
        eid = b * NN + c * ECH + lax.broadcasted_iota(jnp.int32, (1, ECH), 1)
        grf = b * NRF + rf_r
        dest3_ref[0, :, sl] = jnp.where(
            k32, grf * MAXEV + n32.astype(jnp.int32), GG * MAXEV + eid)
        dest33_ref[0, :, sl] = jnp.where(
            k33, grf * TSLOT + n33.astype(jnp.int32), GG * TSLOT + eid)
        cxv = cxr[0, :, sl]
        cyv = cyr[0, :, sl]
        rel = (cxv - (cxv // 2) * 2) + 2 * (cyv - (cyv // 2) * 2)
        rel_ref[0, :, sl] = rel.astype(jnp.float32)
        return 0

    lax.fori_loop(0, NCH, ph2, 0)
    cnt_ref[0, :, :] = cnt


def _binning(cxr, cyr, vr):
    row = pl.BlockSpec((1, 1, NN), lambda i: (i, 0, 0))
    return pl.pallas_call(
        _binning_body,
        grid=(BB,),
        in_specs=[row, row, row],
        out_specs=[row, row, row,
                   pl.BlockSpec((1, 32, 32), lambda i: (i, 0, 0))],
        out_shape=[
            jax.ShapeDtypeStruct((BB, 1, NN), jnp.int32),
            jax.ShapeDtypeStruct((BB, 1, NN), jnp.int32),
            jax.ShapeDtypeStruct((BB, 1, NN), jnp.float32),
            jax.ShapeDtypeStruct((BB, 32, 32), jnp.float32),
        ],
        scratch_shapes=[pltpu.VMEM((1, NN), jnp.float32)],
        compiler_params=pltpu.CompilerParams(
            dimension_semantics=("parallel",)),
    )(cxr, cyr, vr)


def _scatter_feats(dest3, dest33, vx, vrel, vts):
    mesh = plsc.VectorSubcoreMesh(core_axis_name="c", subcore_axis_name="s")

    @functools.partial(
        pl.kernel,
        mesh=mesh,
        out_type=[
            jax.ShapeDtypeStruct((ROWS3,), jnp.float32),
            jax.ShapeDtypeStruct((ROWS3,), jnp.float32),
            jax.ShapeDtypeStruct((ROWS33,), jnp.float32),
        ],
        scratch_types=[
            pltpu.VMEM((NIDX, IDXW), jnp.int32),
            pltpu.VMEM((NIDX, IDXW), jnp.int32),
            pltpu.VMEM((NIDX, IDXW), jnp.float32),
            pltpu.VMEM((NIDX, IDXW), jnp.float32),
            pltpu.VMEM((NIDX, IDXW), jnp.float32),
            pltpu.SemaphoreType.DMA,
        ],
    )
    def k(d3_hbm, d33_hbm, x_hbm, rel_hbm, ts_hbm,
          ox_hbm, orel_hbm, ots_hbm,
          i3_v, i33_v, x_v, rel_v, ts_v, sem):
        wid = lax.axis_index("s") * 2 + lax.axis_index("c")
        pltpu.sync_copy(d3_hbm.at[wid], i3_v)
        pltpu.sync_copy(d33_hbm.at[wid], i33_v)
        pltpu.sync_copy(x_hbm.at[wid], x_v)
        pltpu.sync_copy(rel_hbm.at[wid], rel_v)
        pltpu.sync_copy(ts_hbm.at[wid], ts_v)

        def body(j, _):
            c0 = pltpu.async_copy(x_v.at[j], ox_hbm.at[i3_v.at[j]], sem)
            c1 = pltpu.async_copy(rel_v.at[j], orel_hbm.at[i3_v.at[j]], sem)
            c2 = pltpu.async_copy(ts_v.at[j], ots_hbm.at[i33_v.at[j]], sem)
            c0.wait()
            c1.wait()
            c2.wait()
            return 0

        lax.fori_loop(0, NIDX, body, 0)

    return k(dest3, dest33, vx, vrel, vts)


def _lstm_body(tx, trel, tse, cnt_ref, wx_ref, wh_ref, bias_ref, out_ref):
    gb = out_ref.shape[0]
    wh = wh_ref[...]
    bias = bias_ref[...]
    cntv = cnt_ref[0]
    lens = jnp.minimum(cntv, float(MAXEV))
    has_extra = cntv > float(MAXEV)
    ts_all = tse[...]
    d_hi = ts_all[:, 1:TSLOT] - ts_all[:, 0:MAXEV]
    d_lo = jnp.concatenate(
        [jnp.zeros((gb, 1), jnp.float32), d_hi[:, 0:MAXEV - 1]], axis=1)
    delay = jnp.where(has_extra, d_hi, d_lo)
    step = lax.broadcasted_iota(jnp.int32, (1, MAXEV), 1).astype(jnp.float32)
    tmask = step < lens
    maxd = jnp.max(jnp.where(tmask, delay, 0.0), axis=1, keepdims=True)
    nd = delay / jnp.maximum(maxd, 1e-9)
    relv = trel[...]
    rely = jnp.floor(relv * 0.5)
    relx = relv - 2.0 * rely
    xv = tx[...]
    w0 = wx_ref[0:1, :]
    w1 = wx_ref[1:2, :]
    w2 = wx_ref[2:3, :]
    w3 = wx_ref[3:4, :]
    h = jnp.zeros((gb, 16), jnp.float32)
    c = jnp.zeros((gb, 16), jnp.float32)
    for t in range(MAXEV):
        mask = lens > float(t)
        xg = (xv[:, t:t + 1] * w0 + relx[:, t:t + 1] * w1
              + rely[:, t:t + 1] * w2 + nd[:, t:t + 1] * w3)
        gates = (xg + jnp.dot(h, wh, preferred_element_type=jnp.float32)
                 + bias)
        i = 1.0 / (1.0 + jnp.exp(-gates[:, 0:16]))
        f = 1.0 / (1.0 + jnp.exp(-gates[:, 16:32]))
        g = jnp.tanh(gates[:, 32:48])
        o = 1.0 / (1.0 + jnp.exp(-gates[:, 48:64]))
        c_new = f * c + i * g
        h_new = o * jnp.tanh(c_new)
        h = jnp.where(mask, h_new, h)
        c = jnp.where(mask, c_new, c)
    out_ref[...] = h


def _lstm(t_x, t_rel, t_ts, cnt3, wx8, wh, bias):
    gblk = 1024
    return pl.pallas_call(
        _lstm_body,
        grid=(GG // gblk,),
        in_specs=[
            pl.BlockSpec((gblk, MAXEV), lambda i: (i, 0)),
            pl.BlockSpec((gblk, MAXEV), lambda i: (i, 0)),
            pl.BlockSpec((gblk, TSLOT), lambda i: (i, 0)),
            pl.BlockSpec((1, gblk, 1), lambda i: (i, 0, 0)),
            pl.BlockSpec((8, 64), lambda i: (0, 0)),
            pl.BlockSpec((16, 64), lambda i: (0, 0)),
            pl.BlockSpec((1, 64), lambda i: (0, 0)),
        ],
        out_specs=pl.BlockSpec((gblk, 16), lambda i: (i, 0)),
        out_shape=jax.ShapeDtypeStruct((GG, 16), jnp.float32),
        compiler_params=pltpu.CompilerParams(
            dimension_semantics=("parallel",)),
    )(t_x, t_rel, t_ts, cnt3, wx8, wh, bias)


def kernel(x, coords, ts, lengths, W_ih, W_hh, b_ih, b_hh):
    cx = coords[..., 0].astype(jnp.int32)
    cy = coords[..., 1].astype(jnp.int32)
    valid = (jnp.arange(NN)[None, :] < lengths[:, None]).astype(jnp.float32)
    xs = x[..., 0]
    tss = ts[..., 0]
    row = lambda a: a.reshape(BB, 1, NN)
    dest3, dest33, rel, cnt = _binning(row(cx), row(cy), row(valid))
    wrk = lambda a: a.reshape(NW, NIDX, IDXW)
    t_x, t_rel, t_ts = _scatter_feats(
        wrk(dest3), wrk(dest33), wrk(xs), wrk(rel), wrk(tss))
    wx8 = jnp.pad(W_ih.T, ((0, 4), (0, 0)))
    bias = (b_ih + b_hh).reshape(1, 64)
    h = _lstm(t_x[:GG * MAXEV].reshape(GG, MAXEV),
              t_rel[:GG * MAXEV].reshape(GG, MAXEV),
              t_ts[:GG * TSLOT].reshape(GG, TSLOT),
              cnt.reshape(BB, NRF, 1), wx8, W_hh.T, bias)
    return h.reshape(BB, 32, 32, 16)
